# baseline (speedup 1.0000x reference)
import functools
import math

import jax
import jax.numpy as jnp
from jax import lax
from jax.experimental import pallas as pl
from jax.experimental.pallas import tpu as pltpu
from jax.experimental.pallas import tpu_sc as plsc

PAD = 0
SM = 0.1

NC = 2
NS = 16
L = 16
NW = NC * NS


def _sc_gather_corr(lp, tflat, n_rows, vocab, c_const, w_t):
    rows_per_w = n_rows // NW
    mesh = plsc.VectorSubcoreMesh(core_axis_name="c", subcore_axis_name="s")

    @functools.partial(
        pl.kernel,
        out_type=jax.ShapeDtypeStruct((n_rows,), jnp.float32),
        mesh=mesh,
        scratch_types=[
            pltpu.VMEM((NW, L), jnp.int32),
            pltpu.VMEM((L, 8, 128), jnp.float32),
            pltpu.VMEM((L,), jnp.float32),
            pltpu.SemaphoreType.DMA,
        ],
    )
    def body(lp_hbm, trg_hbm, out_hbm, tall_v, win_v, corr_v, sem):
        wid = lax.axis_index("s") * NC + lax.axis_index("c")
        base = wid * rows_per_w
        pltpu.sync_copy(trg_hbm, tall_v)
        t = tall_v[0, :]
        for k in range(1, NW):
            t = jnp.where(wid == k, tall_v[k, :], t)
        descs = []
        for i in range(L):
            c0 = pl.multiple_of(
                lax.shift_left(lax.shift_right_logical(t[i], 7), 7), 128)
            r0 = base + (i // 8) * 8
            descs.append(pltpu.async_copy(
                lp_hbm.at[pl.ds(r0, 8), pl.ds(c0, 128)], win_v.at[i], sem))
        for d in descs:
            d.wait()
        lane = lax.iota(jnp.int32, L)
        dnums = lax.GatherDimensionNumbers(
            offset_dims=(), collapsed_slice_dims=(0,), start_index_map=(0,))
        tval = jnp.zeros((L,), jnp.float32)
        for i in range(L):
            off = lax.bitwise_and(t[i], 127)
            sel = jnp.zeros((L,), jnp.float32)
            for k in range(128 // L):
                hit = (lane + k * L) == off
                sel = jnp.where(hit, win_v[i, i % 8, k * L:(k + 1) * L], sel)
            g = lax.gather(sel, jnp.full((L, 1), lax.bitwise_and(off, L - 1),
                                         jnp.int32),
                           dnums, slice_sizes=(1,),
                           mode=lax.GatherScatterMode.PROMISE_IN_BOUNDS)
            tval = jnp.where(lane == i, g, tval)
        corr = jnp.where(t != PAD,
                         jnp.float32(c_const) + jnp.float32(w_t) * tval,
                         jnp.float32(0.0))
        corr_v[...] = corr
        pltpu.sync_copy(corr_v, out_hbm.at[pl.ds(base, rows_per_w)])

    return body(lp, tflat)


def _combine_body(d_ref, corr_ref, out_ref):
    out_ref[0, 0] = d_ref[0, 0] + jnp.sum(corr_ref[...])


def _tc_body(nb, vb, vocab, eps, lp_ref, trg_ref, out_ref, acc_ref):
    j = pl.program_id(0)

    def accum(x):
        s = x[:, 0:128]
        for k in range(1, vb // 128):
            s = s + x[:, 128 * k:128 * (k + 1)]
        acc_ref[...] += s

    @pl.when(j == 0)
    def _():
        acc_ref[...] = jnp.zeros_like(acc_ref)
        m = (trg_ref[...] != PAD).astype(jnp.float32)
        zsum = jnp.sum(lp_ref[:, 0:1].reshape(NW, L) * m)
        out_ref[0, 0] = jnp.float32(eps) * zsum

    @pl.when(j < nb - 1)
    def _():
        accum(lp_ref[...])

    @pl.when(j == nb - 1)
    def _():
        col = (nb - 1) * vb + lax.broadcasted_iota(jnp.int32, lp_ref.shape, 1)
        accum(jnp.where(col < vocab, lp_ref[...], 0.0))
        m = (trg_ref[...] != PAD).astype(jnp.float32)
        masked = acc_ref[...].reshape(NW, L, 128) * m[:, :, None]
        out_ref[0, 0] -= jnp.float32(eps) * jnp.sum(masked)


def kernel(log_probs, trg):
    vocab = log_probs.shape[-1]
    lp = log_probs.reshape(-1, vocab)
    n = lp.shape[0]
    trg2 = trg.reshape(NW, L).astype(jnp.int32)

    eps = SM / (vocab - 2)
    c_const = SM * math.log(eps) + (1.0 - SM) * math.log(1.0 - SM)
    w_t = eps - (1.0 - SM)

    corr = _sc_gather_corr(lp, trg2, n, vocab, c_const, w_t)

    vb = 12288
    nb = (vocab + vb - 1) // vb
    dense = pl.pallas_call(
        functools.partial(_tc_body, nb, vb, vocab, eps),
        grid=(nb,),
        in_specs=[
            pl.BlockSpec((n, vb), lambda j: (0, j)),
            pl.BlockSpec((NW, L), lambda j: (0, 0)),
        ],
        out_specs=pl.BlockSpec(memory_space=pltpu.SMEM),
        out_shape=jax.ShapeDtypeStruct((1, 1), jnp.float32),
        scratch_shapes=[pltpu.VMEM((n, 128), jnp.float32)],
    )(lp, trg2)
    total = pl.pallas_call(
        _combine_body,
        in_specs=[
            pl.BlockSpec(memory_space=pltpu.SMEM),
            pl.BlockSpec((n,), lambda: (0,)),
        ],
        out_specs=pl.BlockSpec(memory_space=pltpu.SMEM),
        out_shape=jax.ShapeDtypeStruct((1, 1), jnp.float32),
    )(dense, corr)
    return total[0, 0]


# speedup vs baseline: 1.0074x; 1.0074x over previous
import functools
import math

import jax
import jax.numpy as jnp
from jax import lax
from jax.experimental import pallas as pl
from jax.experimental.pallas import tpu as pltpu
from jax.experimental.pallas import tpu_sc as plsc

PAD = 0
SM = 0.1

NC = 2
NS = 16
L = 16
NW = NC * NS


def _sc_gather_corr(lp, tflat, n_rows, vocab, c_const, w_t):
    rows_per_w = n_rows // NW
    mesh = plsc.VectorSubcoreMesh(core_axis_name="c", subcore_axis_name="s")

    @functools.partial(
        pl.kernel,
        out_type=jax.ShapeDtypeStruct((n_rows,), jnp.float32),
        mesh=mesh,
        scratch_types=[
            pltpu.VMEM((NW, L), jnp.int32),
            pltpu.VMEM((L, 8, 128), jnp.float32),
            pltpu.VMEM((L,), jnp.float32),
            pltpu.SemaphoreType.DMA,
        ],
    )
    def body(lp_hbm, trg_hbm, out_hbm, tall_v, win_v, corr_v, sem):
        wid = lax.axis_index("s") * NC + lax.axis_index("c")
        base = wid * rows_per_w
        pltpu.sync_copy(trg_hbm, tall_v)
        t = tall_v[0, :]
        for k in range(1, NW):
            t = jnp.where(wid == k, tall_v[k, :], t)
        descs = []
        for i in range(L):
            c0 = pl.multiple_of(
                lax.shift_left(lax.shift_right_logical(t[i], 7), 7), 128)
            r0 = base + (i // 8) * 8
            descs.append(pltpu.async_copy(
                lp_hbm.at[pl.ds(r0, 8), pl.ds(c0, 128)], win_v.at[i], sem))
        for d in descs:
            d.wait()
        lane = lax.iota(jnp.int32, L)
        dnums = lax.GatherDimensionNumbers(
            offset_dims=(), collapsed_slice_dims=(0,), start_index_map=(0,))
        tval = jnp.zeros((L,), jnp.float32)
        for i in range(L):
            off = lax.bitwise_and(t[i], 127)
            sel = jnp.zeros((L,), jnp.float32)
            for k in range(128 // L):
                hit = (lane + k * L) == off
                sel = jnp.where(hit, win_v[i, i % 8, k * L:(k + 1) * L], sel)
            g = lax.gather(sel, jnp.full((L, 1), lax.bitwise_and(off, L - 1),
                                         jnp.int32),
                           dnums, slice_sizes=(1,),
                           mode=lax.GatherScatterMode.PROMISE_IN_BOUNDS)
            tval = jnp.where(lane == i, g, tval)
        corr = jnp.where(t != PAD,
                         jnp.float32(c_const) + jnp.float32(w_t) * tval,
                         jnp.float32(0.0))
        corr_v[...] = corr
        pltpu.sync_copy(corr_v, out_hbm.at[pl.ds(base, rows_per_w)])

    return body(lp, tflat)


def _combine_body(d_ref, corr_ref, out_ref):
    out_ref[0, 0] = d_ref[0, 0] + jnp.sum(corr_ref[...])


def _tc_body(nb, vb, vocab, eps, lp_ref, trg_ref, out_ref, acc_ref):
    j = pl.program_id(0)

    def accum(x):
        s = x[:, 0:128]
        for k in range(1, vb // 128):
            s = s + x[:, 128 * k:128 * (k + 1)]
        acc_ref[...] += s

    @pl.when(j == 0)
    def _():
        acc_ref[...] = jnp.zeros_like(acc_ref)
        m = (trg_ref[...] != PAD).astype(jnp.float32)
        zsum = jnp.sum(lp_ref[:, 0:1].reshape(NW, L) * m)
        out_ref[0, 0] = jnp.float32(eps) * zsum

    @pl.when(j < nb - 1)
    def _():
        accum(lp_ref[...])

    @pl.when(j == nb - 1)
    def _():
        col = (nb - 1) * vb + lax.broadcasted_iota(jnp.int32, lp_ref.shape, 1)
        accum(jnp.where(col < vocab, lp_ref[...], 0.0))
        m = (trg_ref[...] != PAD).astype(jnp.float32)
        masked = acc_ref[...].reshape(NW, L, 128) * m[:, :, None]
        out_ref[0, 0] -= jnp.float32(eps) * jnp.sum(masked)


def kernel(log_probs, trg):
    vocab = log_probs.shape[-1]
    lp = log_probs.reshape(-1, vocab)
    n = lp.shape[0]
    trg2 = trg.reshape(NW, L).astype(jnp.int32)

    eps = SM / (vocab - 2)
    c_const = SM * math.log(eps) + (1.0 - SM) * math.log(1.0 - SM)
    w_t = eps - (1.0 - SM)

    corr = _sc_gather_corr(lp, trg2, n, vocab, c_const, w_t)

    vb = 8192
    nb = (vocab + vb - 1) // vb
    dense = pl.pallas_call(
        functools.partial(_tc_body, nb, vb, vocab, eps),
        grid=(nb,),
        in_specs=[
            pl.BlockSpec((n, vb), lambda j: (0, j)),
            pl.BlockSpec((NW, L), lambda j: (0, 0)),
        ],
        out_specs=pl.BlockSpec(memory_space=pltpu.SMEM),
        out_shape=jax.ShapeDtypeStruct((1, 1), jnp.float32),
        scratch_shapes=[pltpu.VMEM((n, 128), jnp.float32)],
    )(lp, trg2)
    total = pl.pallas_call(
        _combine_body,
        in_specs=[
            pl.BlockSpec(memory_space=pltpu.SMEM),
            pl.BlockSpec((n,), lambda: (0,)),
        ],
        out_specs=pl.BlockSpec(memory_space=pltpu.SMEM),
        out_shape=jax.ShapeDtypeStruct((1, 1), jnp.float32),
    )(dense, corr)
    return total[0, 0]
